# baseline (speedup 1.0000x reference)
import functools

import jax
import jax.numpy as jnp
from jax import lax
from jax.experimental import pallas as pl
from jax.experimental.pallas import tpu as pltpu
from jax.experimental.pallas import tpu_sc as plsc

_NQ = 1024
_D = 16
_K = 100
_N = 100000
_NPAD = 100352
_L = 16
_VPG = 32
_HALF = _NPAD // 2
_HG = _HALF // (_VPG * _L)
_CAP = 1024
_TRIG = _CAP - (_VPG * _L + _L)
_BUF = _CAP + 16
_NW = 32
_QPW = _NQ // _NW
_OUT_ROWS = 25
_OUT_ROWS_PAD = 32
_OUT_COLS = 128
_I32_MAX = 2**31 - 1
_NEG_INF = float("-inf")


def _matmul_body(n, blk, q_ref, c_ref, o_ref):
    q = q_ref[...]
    c = c_ref[...]
    s = jax.lax.dot_general(
        q, c, (((1,), (1,)), ((), ())), preferred_element_type=jnp.float32
    )
    col = pl.program_id(0) * blk + jax.lax.broadcasted_iota(
        jnp.int32, (_NQ, blk), 1)
    o_ref[...] = jnp.where(col < n, s, _NEG_INF)


def _scores(queries, candidates):
    n = candidates.shape[0]
    blk = 2048
    grid = _NPAD // blk
    return pl.pallas_call(
        functools.partial(_matmul_body, n, blk),
        grid=(grid,),
        in_specs=[
            pl.BlockSpec((_NQ, _D), lambda i: (0, 0)),
            pl.BlockSpec((blk, _D), lambda i: (i, 0)),
        ],
        out_specs=pl.BlockSpec((_NQ, blk), lambda i: (0, i)),
        out_shape=jax.ShapeDtypeStruct((_NQ, _NPAD), jnp.float32),
    )(queries, candidates)


def _popcnt(m):
    c = plsc.all_reduce_population_count(m)
    if c.ndim:
        c = c[0]
    return c


def _s2i(b):
    return b ^ lax.shift_right_logical(lax.shift_right_arithmetic(b, 31), 1)


def _splat_lane(v, lane):
    idx = jnp.full((_L, 1), lane, jnp.int32)
    return lax.gather(
        v, idx,
        lax.GatherDimensionNumbers(
            offset_dims=(), collapsed_slice_dims=(0,), start_index_map=(0,)),
        (1,), mode=lax.GatherScatterMode.PROMISE_IN_BOUNDS)


def _count_ge(tv_ref, nv, x):
    def body(j, acc):
        s = tv_ref[pl.ds(j * _L, _L)]
        return acc + _popcnt(s >= x)
    return lax.fori_loop(0, nv, body, jnp.int32(0))


def _pad_tail(bv_ref, bi_ref, tv_ref, bcnt):
    bv_ref[pl.ds(bcnt, _L)] = jnp.full((_L,), _NEG_INF, jnp.float32)
    bi_ref[pl.ds(bcnt, _L)] = jnp.full((_L,), _I32_MAX, jnp.int32)
    nv = lax.shift_right_arithmetic(bcnt + jnp.int32(_L - 1), 4)

    def tbody(j, _):
        b = plsc.bitcast(bv_ref[pl.ds(j * _L, _L)], jnp.int32)
        tv_ref[pl.ds(j * _L, _L)] = _s2i(b)
        return jnp.int32(0)

    lax.fori_loop(0, nv, tbody, jnp.int32(0))
    return nv


def _compact(carry, bv_ref, bi_ref, tv_ref):
    bcnt, _ = carry
    nv = _pad_tail(bv_ref, bi_ref, tv_ref, bcnt)

    def cond(st):
        lo, hi, c_lo = st
        return jnp.logical_and(c_lo > _K + 12, hi > lo + 1)

    def step(st):
        lo, hi, c_lo = st
        mid = lo + lax.shift_right_logical(hi - lo, 1)
        c = _count_ge(tv_ref, nv, mid)
        ge = c >= _K
        return (jnp.where(ge, mid, lo), jnp.where(ge, hi, mid),
                jnp.where(ge, c, c_lo))

    lo0 = jnp.int32(-2**31)
    lo, hi, c_lo = lax.while_loop(
        cond, step, (lo0, jnp.int32(_I32_MAX), _count_ge(tv_ref, nv, lo0)))
    t_s = lo

    def sel(j, pos):
        vv = bv_ref[pl.ds(j * _L, _L)]
        vi = bi_ref[pl.ds(j * _L, _L)]
        ts = tv_ref[pl.ds(j * _L, _L)]
        mg = ts >= t_s
        plsc.store_compressed(bv_ref.at[pl.ds(pos, _L)], vv, mask=mg)
        plsc.store_compressed(bi_ref.at[pl.ds(pos, _L)], vi, mask=mg)
        return pos + _popcnt(mg)

    newcnt = lax.fori_loop(0, nv, sel, jnp.int32(0))
    thr = plsc.bitcast(_s2i(jnp.full((_L,), t_s)), jnp.float32)[0]
    return newcnt, thr


def _finalize(carry, bv_ref, bi_ref, tv_ref, ti_ref):
    bcnt, _ = carry
    nv = _pad_tail(bv_ref, bi_ref, tv_ref, bcnt)

    def cond(st):
        lo, hi = st
        return hi > lo + 1

    def step(st):
        lo, hi = st
        mid = lo + lax.shift_right_logical(hi - lo, 1)
        ge = _count_ge(tv_ref, nv, mid) >= _K
        return jnp.where(ge, mid, lo), jnp.where(ge, hi, mid)

    lo, hi = lax.while_loop(cond, step,
                            (jnp.int32(-2**31), jnp.int32(_I32_MAX)))
    t_s = lo

    def sel(j, st):
        pos, tpos = st
        vv = bv_ref[pl.ds(j * _L, _L)]
        vi = bi_ref[pl.ds(j * _L, _L)]
        ts = tv_ref[pl.ds(j * _L, _L)]
        mg = ts > t_s
        me = ts == t_s
        plsc.store_compressed(bv_ref.at[pl.ds(pos, _L)], vv, mask=mg)
        plsc.store_compressed(bi_ref.at[pl.ds(pos, _L)], vi, mask=mg)
        plsc.store_compressed(ti_ref.at[pl.ds(tpos, _L)], vi, mask=me)
        return pos + _popcnt(mg), tpos + _popcnt(me)

    ngt, _ = lax.fori_loop(0, nv, sel, (jnp.int32(0), jnp.int32(0)))

    n_eq = jnp.int32(_K) - ngt
    tfv = plsc.bitcast(_s2i(jnp.full((_L,), t_s)), jnp.float32)
    nv2 = lax.shift_right_arithmetic(n_eq + jnp.int32(_L - 1), 4)

    def tie(j, _):
        idxs = ti_ref[pl.ds(j * _L, _L)]
        bi_ref[pl.ds(ngt + j * _L, _L)] = idxs
        bv_ref[pl.ds(ngt + j * _L, _L)] = tfv
        return jnp.int32(0)

    lax.fori_loop(0, nv2, tie, jnp.int32(0))
    return jnp.int32(_K), tfv[0]


def _topk_body(scores_hbm, ident_hbm, out_vals_hbm, out_ids_hbm,
               b0_ref, b1_ref, bv_ref, bi_ref, tv_ref, ti_ref,
               vals_st, idx_st, ids_st, sem0, sem1, semg):
    nc = 2
    wid = lax.axis_index("s") * nc + lax.axis_index("c")
    lane_iota = lax.iota(jnp.int32, _L)
    neg_vec = jnp.full((_L,), _NEG_INF, jnp.float32)

    def make_group(buf_ref, half_off):
        def group(g, carry):
            base = g * (_VPG * _L)
            bcnt, thr = carry
            vs = [buf_ref[pl.ds(base + u * _L, _L)] for u in range(_VPG)]
            ms = [v > thr for v in vs]
            tree = list(ms)
            while len(tree) > 1:
                tree = [jnp.logical_or(a, b) for a, b in zip(tree[::2], tree[1::2])]
            any_m = tree[0]

            def hit(c):
                bcnt, thr = c
                cs = [_popcnt(ms[u]) for u in range(_VPG)]
                offs = []
                acc = bcnt
                for u in range(_VPG):
                    offs.append(acc)
                    acc = acc + cs[u]
                for u in range(_VPG):
                    plsc.store_compressed(
                        bv_ref.at[pl.ds(offs[u], _L)], vs[u], mask=ms[u])
                    plsc.store_compressed(
                        bi_ref.at[pl.ds(offs[u], _L)],
                        half_off + base + u * _L + lane_iota, mask=ms[u])
                return lax.cond(acc > _TRIG,
                                lambda c2: _compact(c2, bv_ref, bi_ref, tv_ref),
                                lambda c2: c2, (acc, thr))

            return lax.cond(_popcnt(any_m) > 0, hit, lambda c: c, (bcnt, thr))
        return group

    group0 = make_group(b0_ref, 0)
    group1 = make_group(b1_ref, _HALF)

    first = wid * _QPW
    pltpu.async_copy(scores_hbm.at[first, pl.ds(0, _HALF)], b0_ref, sem0)
    pltpu.async_copy(scores_hbm.at[first, pl.ds(_HALF, _HALF)], b1_ref, sem1)

    def per_q(q, _):
        grow = wid * _QPW + q
        grow_next = wid * _QPW + jnp.minimum(q + 1, jnp.int32(_QPW - 1))

        pltpu.make_async_copy(
            scores_hbm.at[grow, pl.ds(0, _HALF)], b0_ref, sem0).wait()
        carry = lax.fori_loop(
            0, _HG, group0, (jnp.int32(0), jnp.float32(_NEG_INF)))

        @pl.when(q < _QPW - 1)
        def _():
            pltpu.async_copy(
                scores_hbm.at[grow_next, pl.ds(0, _HALF)], b0_ref, sem0)

        pltpu.make_async_copy(
            scores_hbm.at[grow, pl.ds(_HALF, _HALF)], b1_ref, sem1).wait()
        carry = lax.fori_loop(0, _HG, group1, carry)

        @pl.when(q < _QPW - 1)
        def _():
            pltpu.async_copy(
                scores_hbm.at[grow_next, pl.ds(_HALF, _HALF)], b1_ref, sem1)

        _finalize(carry, bv_ref, bi_ref, tv_ref, ti_ref)

        bv_ref[pl.ds(_K, _L)] = neg_vec
        bi_ref[pl.ds(_K, _L)] = jnp.full((_L,), _I32_MAX, jnp.int32)

        def rank_body(e, _):
            jv = lax.shift_right_arithmetic(e, 4)
            lane = jnp.bitwise_and(e, jnp.int32(_L - 1))
            vv = bv_ref[pl.ds(jv * _L, _L)]
            vi = bi_ref[pl.ds(jv * _L, _L)]
            sv = _splat_lane(vv, lane)
            si = _splat_lane(vi, lane)
            r = jnp.int32(0)
            for j in range(7):
                cv = bv_ref[pl.ds(j * _L, _L)]
                ci = bi_ref[pl.ds(j * _L, _L)]
                m = jnp.logical_or(
                    cv > sv, jnp.logical_and(cv == sv, ci < si))
                r = r + _popcnt(m)
            pos = q * _K + r
            hi_v = jnp.full((_L,), lax.shift_right_arithmetic(pos, 7))
            lo_v = jnp.full((_L,), jnp.bitwise_and(pos, jnp.int32(127)))
            m0 = lane_iota == 0
            plsc.store_scatter(vals_st, [hi_v, lo_v], sv, mask=m0)
            plsc.store_scatter(idx_st, [hi_v, lo_v], si, mask=m0)
            return jnp.int32(0)

        lax.fori_loop(0, _K, rank_body, jnp.int32(0))
        return jnp.int32(0)

    lax.fori_loop(0, _QPW, per_q, jnp.int32(0))

    copies = [
        pltpu.async_copy(ident_hbm.at[idx_st.at[j]], ids_st.at[j], semg)
        for j in range(_OUT_ROWS)
    ]
    for c in copies:
        c.wait()

    pltpu.sync_copy(vals_st, out_vals_hbm.at[wid])
    pltpu.sync_copy(ids_st, out_ids_hbm.at[wid])


@functools.partial(
    pl.kernel,
    out_type=(
        jax.ShapeDtypeStruct((_NW, _OUT_ROWS_PAD, _OUT_COLS), jnp.float32),
        jax.ShapeDtypeStruct((_NW, _OUT_ROWS_PAD, _OUT_COLS), jnp.int32),
    ),
    mesh=plsc.VectorSubcoreMesh(core_axis_name="c", subcore_axis_name="s"),
    compiler_params=pltpu.CompilerParams(needs_layout_passes=False),
    scratch_types=(
        pltpu.VMEM((_HALF,), jnp.float32),
        pltpu.VMEM((_HALF,), jnp.float32),
        pltpu.VMEM((_BUF,), jnp.float32),
        pltpu.VMEM((_BUF,), jnp.int32),
        pltpu.VMEM((_BUF,), jnp.int32),
        pltpu.VMEM((_BUF,), jnp.int32),
        pltpu.VMEM((_OUT_ROWS_PAD, _OUT_COLS), jnp.float32),
        pltpu.VMEM((_OUT_ROWS_PAD, _OUT_COLS), jnp.int32),
        pltpu.VMEM((_OUT_ROWS_PAD, _OUT_COLS), jnp.int32),
        pltpu.SemaphoreType.DMA,
        pltpu.SemaphoreType.DMA,
        pltpu.SemaphoreType.DMA,
    ),
)
def _sc_topk(scores_hbm, ident_hbm, out_vals_hbm, out_ids_hbm,
             b0_ref, b1_ref, bv_ref, bi_ref, tv_ref, ti_ref,
             vals_st, idx_st, ids_st, sem0, sem1, semg):
    _topk_body(scores_hbm, ident_hbm, out_vals_hbm, out_ids_hbm,
               b0_ref, b1_ref, bv_ref, bi_ref, tv_ref, ti_ref,
               vals_st, idx_st, ids_st, sem0, sem1, semg)


@jax.jit
def _impl(queries, candidates, ident32):
    scores = _scores(queries, candidates)
    vals, ids = _sc_topk(scores, ident32)
    vals = vals[:, :_OUT_ROWS, :].reshape(_NQ, _K)
    ids = ids[:, :_OUT_ROWS, :].reshape(_NQ, _K)
    return vals, ids


def kernel(queries, candidates, identifiers, num_candidates):
    ident32 = identifiers.astype(jnp.int32)
    values, top_ids = _impl(queries, candidates, ident32)
    zero_dep = num_candidates - num_candidates
    return values, top_ids.astype(identifiers.dtype) + zero_dep


# speedup vs baseline: 1.0020x; 1.0020x over previous
import functools

import jax
import jax.numpy as jnp
from jax import lax
from jax.experimental import pallas as pl
from jax.experimental.pallas import tpu as pltpu
from jax.experimental.pallas import tpu_sc as plsc

_NQ = 1024
_D = 16
_K = 100
_N = 100000
_NPAD = 100352
_L = 16
_VPG = 32
_HALF = _NPAD // 2
_HG = _HALF // (_VPG * _L)
_CAP = 1024
_TRIG = _CAP - (_VPG * _L + _L)
_BUF = _CAP + 16
_NW = 32
_QPW = _NQ // _NW
_OUT_ROWS = 25
_OUT_ROWS_PAD = 32
_OUT_COLS = 128
_I32_MAX = 2**31 - 1
_NEG_INF = float("-inf")


def _matmul_body(n, blk, q_ref, c_ref, o_ref):
    q = q_ref[...]
    c = c_ref[...]
    s = jax.lax.dot_general(
        q, c, (((1,), (1,)), ((), ())), preferred_element_type=jnp.float32
    )
    col = pl.program_id(0) * blk + jax.lax.broadcasted_iota(
        jnp.int32, (_NQ, blk), 1)
    o_ref[...] = jnp.where(col < n, s, _NEG_INF)


def _scores(queries, candidates):
    n = candidates.shape[0]
    blk = 2048
    grid = _NPAD // blk
    return pl.pallas_call(
        functools.partial(_matmul_body, n, blk),
        grid=(grid,),
        in_specs=[
            pl.BlockSpec((_NQ, _D), lambda i: (0, 0)),
            pl.BlockSpec((blk, _D), lambda i: (i, 0)),
        ],
        out_specs=pl.BlockSpec((_NQ, blk), lambda i: (0, i)),
        out_shape=jax.ShapeDtypeStruct((_NQ, _NPAD), jnp.float32),
    )(queries, candidates)


def _popcnt(m):
    c = plsc.all_reduce_population_count(m)
    if c.ndim:
        c = c[0]
    return c


def _s2i(b):
    return b ^ lax.shift_right_logical(lax.shift_right_arithmetic(b, 31), 1)


def _splat_lane(v, lane):
    idx = jnp.full((_L, 1), lane, jnp.int32)
    return lax.gather(
        v, idx,
        lax.GatherDimensionNumbers(
            offset_dims=(), collapsed_slice_dims=(0,), start_index_map=(0,)),
        (1,), mode=lax.GatherScatterMode.PROMISE_IN_BOUNDS)


def _count_ge(tv_ref, nv, x):
    def body(j, acc):
        s = tv_ref[pl.ds(j * _L, _L)]
        return acc + _popcnt(s >= x)
    return lax.fori_loop(0, nv, body, jnp.int32(0))


def _pad_tail(bv_ref, bi_ref, tv_ref, bcnt):
    bv_ref[pl.ds(bcnt, _L)] = jnp.full((_L,), _NEG_INF, jnp.float32)
    bi_ref[pl.ds(bcnt, _L)] = jnp.full((_L,), _I32_MAX, jnp.int32)
    nv = lax.shift_right_arithmetic(bcnt + jnp.int32(_L - 1), 4)

    def tbody(j, _):
        b = plsc.bitcast(bv_ref[pl.ds(j * _L, _L)], jnp.int32)
        tv_ref[pl.ds(j * _L, _L)] = _s2i(b)
        return jnp.int32(0)

    lax.fori_loop(0, nv, tbody, jnp.int32(0))
    return nv


def _compact(carry, bv_ref, bi_ref, tv_ref):
    bcnt, _ = carry
    nv = _pad_tail(bv_ref, bi_ref, tv_ref, bcnt)

    def cond(st):
        lo, hi, c_lo = st
        return jnp.logical_and(c_lo > _K + 12, hi > lo + 1)

    def step(st):
        lo, hi, c_lo = st
        mid = lo + lax.shift_right_logical(hi - lo, 1)
        c = _count_ge(tv_ref, nv, mid)
        ge = c >= _K
        return (jnp.where(ge, mid, lo), jnp.where(ge, hi, mid),
                jnp.where(ge, c, c_lo))

    lo0 = jnp.int32(-2**31)
    lo, hi, c_lo = lax.while_loop(
        cond, step, (lo0, jnp.int32(_I32_MAX), _count_ge(tv_ref, nv, lo0)))
    t_s = lo

    def sel(j, pos):
        vv = bv_ref[pl.ds(j * _L, _L)]
        vi = bi_ref[pl.ds(j * _L, _L)]
        ts = tv_ref[pl.ds(j * _L, _L)]
        mg = ts >= t_s
        plsc.store_compressed(bv_ref.at[pl.ds(pos, _L)], vv, mask=mg)
        plsc.store_compressed(bi_ref.at[pl.ds(pos, _L)], vi, mask=mg)
        return pos + _popcnt(mg)

    newcnt = lax.fori_loop(0, nv, sel, jnp.int32(0))
    thr = plsc.bitcast(_s2i(jnp.full((_L,), t_s)), jnp.float32)[0]
    return newcnt, thr


def _finalize(carry, bv_ref, bi_ref, tv_ref, ti_ref):
    bcnt, _ = carry
    nv = _pad_tail(bv_ref, bi_ref, tv_ref, bcnt)

    def cond(st):
        lo, hi = st
        return hi > lo + 1

    def step(st):
        lo, hi = st
        mid = lo + lax.shift_right_logical(hi - lo, 1)
        ge = _count_ge(tv_ref, nv, mid) >= _K
        return jnp.where(ge, mid, lo), jnp.where(ge, hi, mid)

    lo, hi = lax.while_loop(cond, step,
                            (jnp.int32(-2**31), jnp.int32(_I32_MAX)))
    t_s = lo

    def sel(j, st):
        pos, tpos = st
        vv = bv_ref[pl.ds(j * _L, _L)]
        vi = bi_ref[pl.ds(j * _L, _L)]
        ts = tv_ref[pl.ds(j * _L, _L)]
        mg = ts > t_s
        me = ts == t_s
        plsc.store_compressed(bv_ref.at[pl.ds(pos, _L)], vv, mask=mg)
        plsc.store_compressed(bi_ref.at[pl.ds(pos, _L)], vi, mask=mg)
        plsc.store_compressed(ti_ref.at[pl.ds(tpos, _L)], vi, mask=me)
        return pos + _popcnt(mg), tpos + _popcnt(me)

    ngt, _ = lax.fori_loop(0, nv, sel, (jnp.int32(0), jnp.int32(0)))

    n_eq = jnp.int32(_K) - ngt
    tfv = plsc.bitcast(_s2i(jnp.full((_L,), t_s)), jnp.float32)
    nv2 = lax.shift_right_arithmetic(n_eq + jnp.int32(_L - 1), 4)

    def tie(j, _):
        idxs = ti_ref[pl.ds(j * _L, _L)]
        bi_ref[pl.ds(ngt + j * _L, _L)] = idxs
        bv_ref[pl.ds(ngt + j * _L, _L)] = tfv
        return jnp.int32(0)

    lax.fori_loop(0, nv2, tie, jnp.int32(0))
    return jnp.int32(_K), tfv[0]


def _topk_body(scores_hbm, ident_hbm, out_vals_hbm, out_ids_hbm,
               b0_ref, b1_ref, bv_ref, bi_ref, tv_ref, ti_ref,
               vals_st, idx_st, ids_st, sem0, sem1, semg):
    nc = 2
    wid = lax.axis_index("s") * nc + lax.axis_index("c")
    lane_iota = lax.iota(jnp.int32, _L)
    neg_vec = jnp.full((_L,), _NEG_INF, jnp.float32)

    def make_group(buf_ref, half_off):
        def group(g, carry):
            base = g * (_VPG * _L)
            bcnt, thr = carry
            vs = [buf_ref[pl.ds(base + u * _L, _L)] for u in range(_VPG)]
            ms = [v > thr for v in vs]
            tree = list(ms)
            while len(tree) > 1:
                tree = [jnp.logical_or(a, b) for a, b in zip(tree[::2], tree[1::2])]
            any_m = tree[0]

            def hit(c):
                bcnt, thr = c
                cs = [_popcnt(ms[u]) for u in range(_VPG)]
                offs = []
                acc = bcnt
                for u in range(_VPG):
                    offs.append(acc)
                    acc = acc + cs[u]
                for u in range(_VPG):
                    plsc.store_compressed(
                        bv_ref.at[pl.ds(offs[u], _L)], vs[u], mask=ms[u])
                    plsc.store_compressed(
                        bi_ref.at[pl.ds(offs[u], _L)],
                        half_off + base + u * _L + lane_iota, mask=ms[u])
                return lax.cond(acc > _TRIG,
                                lambda c2: _compact(c2, bv_ref, bi_ref, tv_ref),
                                lambda c2: c2, (acc, thr))

            return lax.cond(_popcnt(any_m) > 0, hit, lambda c: c, (bcnt, thr))
        return group

    group0 = make_group(b0_ref, 0)
    group1 = make_group(b1_ref, _HALF)

    first = wid * _QPW
    pltpu.async_copy(scores_hbm.at[first, pl.ds(0, _HALF)], b0_ref, sem0)
    pltpu.async_copy(scores_hbm.at[first, pl.ds(_HALF, _HALF)], b1_ref, sem1)

    def per_q(q, _):
        grow = wid * _QPW + q
        grow_next = wid * _QPW + jnp.minimum(q + 1, jnp.int32(_QPW - 1))

        def duo0(k, c):
            return group0(2 * k + 1, group0(2 * k, c))

        def duo1(k, c):
            return group1(2 * k + 1, group1(2 * k, c))

        pltpu.make_async_copy(
            scores_hbm.at[grow, pl.ds(0, _HALF)], b0_ref, sem0).wait()
        carry = lax.fori_loop(
            0, _HG // 2, duo0, (jnp.int32(0), jnp.float32(_NEG_INF)))

        @pl.when(q < _QPW - 1)
        def _():
            pltpu.async_copy(
                scores_hbm.at[grow_next, pl.ds(0, _HALF)], b0_ref, sem0)

        pltpu.make_async_copy(
            scores_hbm.at[grow, pl.ds(_HALF, _HALF)], b1_ref, sem1).wait()
        carry = lax.fori_loop(0, _HG // 2, duo1, carry)

        @pl.when(q < _QPW - 1)
        def _():
            pltpu.async_copy(
                scores_hbm.at[grow_next, pl.ds(_HALF, _HALF)], b1_ref, sem1)

        _finalize(carry, bv_ref, bi_ref, tv_ref, ti_ref)

        bv_ref[pl.ds(_K, _L)] = neg_vec
        bi_ref[pl.ds(_K, _L)] = jnp.full((_L,), _I32_MAX, jnp.int32)

        def rank_body(e, _):
            jv = lax.shift_right_arithmetic(e, 4)
            lane = jnp.bitwise_and(e, jnp.int32(_L - 1))
            vv = bv_ref[pl.ds(jv * _L, _L)]
            vi = bi_ref[pl.ds(jv * _L, _L)]
            sv = _splat_lane(vv, lane)
            si = _splat_lane(vi, lane)
            r = jnp.int32(0)
            for j in range(7):
                cv = bv_ref[pl.ds(j * _L, _L)]
                ci = bi_ref[pl.ds(j * _L, _L)]
                m = jnp.logical_or(
                    cv > sv, jnp.logical_and(cv == sv, ci < si))
                r = r + _popcnt(m)
            pos = q * _K + r
            hi_v = jnp.full((_L,), lax.shift_right_arithmetic(pos, 7))
            lo_v = jnp.full((_L,), jnp.bitwise_and(pos, jnp.int32(127)))
            m0 = lane_iota == 0
            plsc.store_scatter(vals_st, [hi_v, lo_v], sv, mask=m0)
            plsc.store_scatter(idx_st, [hi_v, lo_v], si, mask=m0)
            return jnp.int32(0)

        lax.fori_loop(0, _K, rank_body, jnp.int32(0))
        return jnp.int32(0)

    lax.fori_loop(0, _QPW, per_q, jnp.int32(0))

    copies = [
        pltpu.async_copy(ident_hbm.at[idx_st.at[j]], ids_st.at[j], semg)
        for j in range(_OUT_ROWS)
    ]
    for c in copies:
        c.wait()

    pltpu.sync_copy(vals_st, out_vals_hbm.at[wid])
    pltpu.sync_copy(ids_st, out_ids_hbm.at[wid])


@functools.partial(
    pl.kernel,
    out_type=(
        jax.ShapeDtypeStruct((_NW, _OUT_ROWS_PAD, _OUT_COLS), jnp.float32),
        jax.ShapeDtypeStruct((_NW, _OUT_ROWS_PAD, _OUT_COLS), jnp.int32),
    ),
    mesh=plsc.VectorSubcoreMesh(core_axis_name="c", subcore_axis_name="s"),
    compiler_params=pltpu.CompilerParams(needs_layout_passes=False),
    scratch_types=(
        pltpu.VMEM((_HALF,), jnp.float32),
        pltpu.VMEM((_HALF,), jnp.float32),
        pltpu.VMEM((_BUF,), jnp.float32),
        pltpu.VMEM((_BUF,), jnp.int32),
        pltpu.VMEM((_BUF,), jnp.int32),
        pltpu.VMEM((_BUF,), jnp.int32),
        pltpu.VMEM((_OUT_ROWS_PAD, _OUT_COLS), jnp.float32),
        pltpu.VMEM((_OUT_ROWS_PAD, _OUT_COLS), jnp.int32),
        pltpu.VMEM((_OUT_ROWS_PAD, _OUT_COLS), jnp.int32),
        pltpu.SemaphoreType.DMA,
        pltpu.SemaphoreType.DMA,
        pltpu.SemaphoreType.DMA,
    ),
)
def _sc_topk(scores_hbm, ident_hbm, out_vals_hbm, out_ids_hbm,
             b0_ref, b1_ref, bv_ref, bi_ref, tv_ref, ti_ref,
             vals_st, idx_st, ids_st, sem0, sem1, semg):
    _topk_body(scores_hbm, ident_hbm, out_vals_hbm, out_ids_hbm,
               b0_ref, b1_ref, bv_ref, bi_ref, tv_ref, ti_ref,
               vals_st, idx_st, ids_st, sem0, sem1, semg)


@jax.jit
def _impl(queries, candidates, ident32):
    scores = _scores(queries, candidates)
    vals, ids = _sc_topk(scores, ident32)
    vals = vals[:, :_OUT_ROWS, :].reshape(_NQ, _K)
    ids = ids[:, :_OUT_ROWS, :].reshape(_NQ, _K)
    return vals, ids


def kernel(queries, candidates, identifiers, num_candidates):
    ident32 = identifiers.astype(jnp.int32)
    values, top_ids = _impl(queries, candidates, ident32)
    zero_dep = num_candidates - num_candidates
    return values, top_ids.astype(identifiers.dtype) + zero_dep
